# baseline (speedup 1.0000x reference)
import functools

import jax
import jax.numpy as jnp
from jax import lax
from jax.experimental import pallas as pl
from jax.experimental.pallas import tpu as pltpu
from jax.experimental.pallas import tpu_sc as plsc

N = 10000
E = 320000
D = 128

NC, NS = 2, 16
NW = NC * NS
CH = 128
CPT = 80
E_PAD = CPT * CH * NW
NP = 10240
RPT = NP // NS
NBUF = 2

_MESH = plsc.VectorSubcoreMesh(
    core_axis_name="c", subcore_axis_name="s", num_cores=NC, num_subcores=NS
)


def _worker():
    return lax.axis_index("s") * NC + lax.axis_index("c")


def _deg_body(col_hbm, deg0_hbm, deg1_hbm, ones_v, cidx_v, zrow_v, deg_sh, sem):
    c = lax.axis_index("c")
    s = lax.axis_index("s")
    w = _worker()

    def fill(i, _):
        ones_v[pl.ds(i * 16, 16)] = jnp.ones((16,), jnp.float32)
        zrow_v[pl.ds(i * 16, 16)] = jnp.zeros((16,), jnp.float32)
        return 0

    lax.fori_loop(0, CH // 16, fill, 0)

    pltpu.async_copy(col_hbm.at[w], cidx_v, sem).wait()

    def zloop(i, _):
        pltpu.sync_copy(zrow_v, deg_sh.at[pl.ds(s * RPT + i * CH, CH)])
        return 0

    lax.fori_loop(0, RPT // CH, zloop, 0)
    plsc.subcore_barrier()

    def body(j, _):
        pltpu.sync_copy(ones_v, deg_sh.at[cidx_v.at[j]], add=True)
        return 0

    lax.fori_loop(0, CPT, body, 0)
    plsc.subcore_barrier()

    @pl.when(c == 0)
    def _():
        pltpu.sync_copy(deg_sh.at[pl.ds(s * RPT, RPT)],
                        deg0_hbm.at[pl.ds(s * RPT, RPT)])

    @pl.when(c == 1)
    def _():
        pltpu.sync_copy(deg_sh.at[pl.ds(s * RPT, RPT)],
                        deg1_hbm.at[pl.ds(s * RPT, RPT)])


_deg_call = functools.partial(
    pl.kernel,
    out_type=(
        jax.ShapeDtypeStruct((NP,), jnp.float32),
        jax.ShapeDtypeStruct((NP,), jnp.float32),
    ),
    mesh=_MESH,
    scratch_types=[
        pltpu.VMEM((CH,), jnp.float32),
        pltpu.VMEM((CPT, CH), jnp.int32),
        pltpu.VMEM((CH,), jnp.float32),
        pltpu.VMEM_SHARED((NP,), jnp.float32),
        pltpu.SemaphoreType.DMA,
    ],
)(_deg_body)


def _mm_body(x_ref, w_ref, d0_ref, d1_ref, h2_ref, dinv_ref):
    deg = d0_ref[...] + d1_ref[...] + 1.0
    dinv = lax.rsqrt(deg)
    h = jnp.dot(x_ref[...], w_ref[...], preferred_element_type=jnp.float32)
    h2_ref[...] = h * dinv
    dinv_ref[...] = dinv


_MMR = 2000


def _mm_call(x, W, d0, d1):
    grid = N // _MMR
    return pl.pallas_call(
        _mm_body,
        grid=(grid,),
        in_specs=[
            pl.BlockSpec((_MMR, D), lambda i: (i, 0)),
            pl.BlockSpec((D, D), lambda i: (0, 0)),
            pl.BlockSpec((_MMR, 1), lambda i: (i, 0)),
            pl.BlockSpec((_MMR, 1), lambda i: (i, 0)),
        ],
        out_specs=[
            pl.BlockSpec((_MMR, D), lambda i: (i, 0)),
            pl.BlockSpec((_MMR, 1), lambda i: (i, 0)),
        ],
        out_shape=[
            jax.ShapeDtypeStruct((N, D), jnp.float32),
            jax.ShapeDtypeStruct((N, 1), jnp.float32),
        ],
    )(x, W, d0, d1)


SUP = 8
NSU = CPT // SUP


def _scat_body(row_hbm, col_hbm, h2_hbm, p0_hbm, p1_hbm,
               ridx_v, cidx_v, rows_v, acc_sh, semI, semG):
    c = lax.axis_index("c")
    s = lax.axis_index("s")
    w = _worker()

    def load_idx(t, q):
        src_r = row_hbm.at[w, pl.ds(pl.multiple_of(t * SUP, SUP), SUP)]
        src_c = col_hbm.at[w, pl.ds(pl.multiple_of(t * SUP, SUP), SUP)]
        pltpu.async_copy(src_r, ridx_v.at[q], semI.at[q])
        pltpu.async_copy(src_c, cidx_v.at[q], semI.at[q])

    def idx_wait(t, q):
        src_r = row_hbm.at[w, pl.ds(pl.multiple_of(t * SUP, SUP), SUP)]
        src_c = col_hbm.at[w, pl.ds(pl.multiple_of(t * SUP, SUP), SUP)]
        pltpu.make_async_copy(src_r, ridx_v.at[q], semI.at[q]).wait()
        pltpu.make_async_copy(src_c, cidx_v.at[q], semI.at[q]).wait()

    load_idx(0, 0)
    load_idx(1, 1)

    def zb(i, _):
        def zb2(j, _):
            rows_v[0, i, pl.ds(j * 16, 16)] = jnp.zeros((16,), jnp.float32)
            return 0
        lax.fori_loop(0, D // 16, zb2, 0)
        return 0

    lax.fori_loop(0, CH, zb, 0)

    def zloop(k, _):
        pltpu.sync_copy(rows_v.at[0], acc_sh.at[pl.ds(s * RPT + k * CH, CH)])
        return 0

    lax.fori_loop(0, RPT // CH, zloop, 0)
    plsc.subcore_barrier()

    def gather(q, u, buf):
        pltpu.async_copy(h2_hbm.at[ridx_v.at[q, u]], rows_v.at[buf],
                         semG.at[buf])

    def gather_wait(q, u, buf):
        pltpu.make_async_copy(h2_hbm.at[ridx_v.at[q, u]], rows_v.at[buf],
                              semG.at[buf]).wait()

    def scat(q, u, buf):
        pltpu.sync_copy(rows_v.at[buf], acc_sh.at[cidx_v.at[q, u]], add=True)

    def body(tp, _):
        for tt in range(2):
            t = tp * 2 + tt
            idx_wait(t, tt)
            gather(tt, 0, 0)
            gather(tt, 1, 1)
            for u in range(SUP):
                gather_wait(tt, u, u % 2)
                scat(tt, u, u % 2)
                if u + 2 < SUP:
                    gather(tt, u + 2, u % 2)

            @pl.when(tp < NSU // 2 - 1)
            def _():
                load_idx(t + 2, tt)
        return 0

    lax.fori_loop(0, NSU // 2, body, 0)

    plsc.subcore_barrier()

    @pl.when(c == 0)
    def _():
        pltpu.sync_copy(acc_sh.at[pl.ds(s * RPT, RPT)],
                        p0_hbm.at[pl.ds(s * RPT, RPT)])

    @pl.when(c == 1)
    def _():
        pltpu.sync_copy(acc_sh.at[pl.ds(s * RPT, RPT)],
                        p1_hbm.at[pl.ds(s * RPT, RPT)])


_scat_call = functools.partial(
    pl.kernel,
    out_type=(
        jax.ShapeDtypeStruct((NP, D), jnp.float32),
        jax.ShapeDtypeStruct((NP, D), jnp.float32),
    ),
    mesh=_MESH,
    scratch_types=[
        pltpu.VMEM((2, SUP, CH), jnp.int32),
        pltpu.VMEM((2, SUP, CH), jnp.int32),
        pltpu.VMEM((NBUF, CH, D), jnp.float32),
        pltpu.VMEM_SHARED((NP, D), jnp.float32),
        pltpu.SemaphoreType.DMA((2,)),
        pltpu.SemaphoreType.DMA((NBUF,)),
    ],
)(_scat_body)


def _comb_body(p0_ref, p1_ref, h2_ref, dinv_ref, b_ref, out_ref):
    out_ref[...] = (
        dinv_ref[...] * (p0_ref[...] + p1_ref[...] + h2_ref[...]) + b_ref[...]
    )


def _comb_call(p0, p1, h2, dinv, b2):
    grid = N // _MMR
    return pl.pallas_call(
        _comb_body,
        grid=(grid,),
        in_specs=[
            pl.BlockSpec((_MMR, D), lambda i: (i, 0)),
            pl.BlockSpec((_MMR, D), lambda i: (i, 0)),
            pl.BlockSpec((_MMR, D), lambda i: (i, 0)),
            pl.BlockSpec((_MMR, 1), lambda i: (i, 0)),
            pl.BlockSpec((1, D), lambda i: (0, 0)),
        ],
        out_specs=pl.BlockSpec((_MMR, D), lambda i: (i, 0)),
        out_shape=jax.ShapeDtypeStruct((N, D), jnp.float32),
    )(p0, p1, h2, dinv, b2)


def kernel(x, edge_index, W, b):
    row = edge_index[0]
    col = edge_index[1]
    pad = E_PAD - E
    row_p = jnp.concatenate([row, jnp.zeros((pad,), jnp.int32)])
    col_p = jnp.concatenate([col, jnp.full((pad,), N, jnp.int32)])
    row3 = row_p.reshape(NW, CPT, CH)
    col3 = col_p.reshape(NW, CPT, CH)

    deg0, deg1 = _deg_call(col3)
    d0 = deg0[:N, None]
    d1 = deg1[:N, None]
    h2, dinv = _mm_call(x, W, d0, d1)
    p0, p1 = _scat_call(row3, col3, h2)
    b2 = b[None, :]
    return _comb_call(p0, p1, h2, dinv, b2)


# speedup vs baseline: 3.1958x; 3.1958x over previous
import functools

import jax
import jax.numpy as jnp
from jax import lax
from jax.experimental import pallas as pl
from jax.experimental.pallas import tpu as pltpu
from jax.experimental.pallas import tpu_sc as plsc

N = 10000
E = 320000
D = 128

NC, NS = 2, 16
NW = NC * NS
CH = 128
CPT = 80
E_PAD = CPT * CH * NW
NP = 10240
RPT = NP // NS
NBUF = 2

_MESH = plsc.VectorSubcoreMesh(
    core_axis_name="c", subcore_axis_name="s", num_cores=NC, num_subcores=NS
)


def _worker():
    return lax.axis_index("s") * NC + lax.axis_index("c")


def _deg_body(col_hbm, deg0_hbm, deg1_hbm, ones_v, cidx_v, zrow_v, deg_sh, sem):
    c = lax.axis_index("c")
    s = lax.axis_index("s")
    w = _worker()

    def fill(i, _):
        ones_v[pl.ds(i * 16, 16)] = jnp.ones((16,), jnp.float32)
        zrow_v[pl.ds(i * 16, 16)] = jnp.zeros((16,), jnp.float32)
        return 0

    lax.fori_loop(0, CH // 16, fill, 0)

    pltpu.async_copy(col_hbm.at[w], cidx_v, sem).wait()

    def zloop(i, _):
        pltpu.sync_copy(zrow_v, deg_sh.at[pl.ds(s * RPT + i * CH, CH)])
        return 0

    lax.fori_loop(0, RPT // CH, zloop, 0)
    plsc.subcore_barrier()

    def body(j, _):
        pltpu.sync_copy(ones_v, deg_sh.at[cidx_v.at[j]], add=True)
        return 0

    lax.fori_loop(0, CPT, body, 0)
    plsc.subcore_barrier()

    @pl.when(c == 0)
    def _():
        pltpu.sync_copy(deg_sh.at[pl.ds(s * RPT, RPT)],
                        deg0_hbm.at[pl.ds(s * RPT, RPT)])

    @pl.when(c == 1)
    def _():
        pltpu.sync_copy(deg_sh.at[pl.ds(s * RPT, RPT)],
                        deg1_hbm.at[pl.ds(s * RPT, RPT)])


_deg_call = functools.partial(
    pl.kernel,
    out_type=(
        jax.ShapeDtypeStruct((NP,), jnp.float32),
        jax.ShapeDtypeStruct((NP,), jnp.float32),
    ),
    mesh=_MESH,
    scratch_types=[
        pltpu.VMEM((CH,), jnp.float32),
        pltpu.VMEM((CPT, CH), jnp.int32),
        pltpu.VMEM((CH,), jnp.float32),
        pltpu.VMEM_SHARED((NP,), jnp.float32),
        pltpu.SemaphoreType.DMA,
    ],
)(_deg_body)


def _mm_body(x_ref, w_ref, d0_ref, d1_ref, h2_ref, dinv_ref):
    deg = d0_ref[...] + d1_ref[...] + 1.0
    dinv = lax.rsqrt(deg)
    h = jnp.dot(x_ref[...], w_ref[...], preferred_element_type=jnp.float32)
    h2_ref[...] = h * dinv
    dinv_ref[...] = dinv


_MMR = 2000


def _mm_call(x, W, d0, d1):
    grid = N // _MMR
    return pl.pallas_call(
        _mm_body,
        grid=(grid,),
        in_specs=[
            pl.BlockSpec((_MMR, D), lambda i: (i, 0)),
            pl.BlockSpec((D, D), lambda i: (0, 0)),
            pl.BlockSpec((_MMR, 1), lambda i: (i, 0)),
            pl.BlockSpec((_MMR, 1), lambda i: (i, 0)),
        ],
        out_specs=[
            pl.BlockSpec((_MMR, D), lambda i: (i, 0)),
            pl.BlockSpec((_MMR, 1), lambda i: (i, 0)),
        ],
        out_shape=[
            jax.ShapeDtypeStruct((N, D), jnp.float32),
            jax.ShapeDtypeStruct((N, 1), jnp.float32),
        ],
    )(x, W, d0, d1)


SUP = 8
NSU = CPT // SUP


def _scat_body(row_hbm, col_hbm, h2_hbm, p0_hbm, p1_hbm,
               ridx_v, cidx_v, rows_v, acc_sh, semI, semG):
    c = lax.axis_index("c")
    s = lax.axis_index("s")
    w = _worker()

    def load_idx(t, q):
        src_r = row_hbm.at[w, pl.ds(pl.multiple_of(t * SUP, SUP), SUP)]
        src_c = col_hbm.at[w, pl.ds(pl.multiple_of(t * SUP, SUP), SUP)]
        pltpu.async_copy(src_r, ridx_v.at[q], semI.at[q])
        pltpu.async_copy(src_c, cidx_v.at[q], semI.at[q])

    def idx_wait(t, q):
        src_r = row_hbm.at[w, pl.ds(pl.multiple_of(t * SUP, SUP), SUP)]
        src_c = col_hbm.at[w, pl.ds(pl.multiple_of(t * SUP, SUP), SUP)]
        pltpu.make_async_copy(src_r, ridx_v.at[q], semI.at[q]).wait()
        pltpu.make_async_copy(src_c, cidx_v.at[q], semI.at[q]).wait()

    load_idx(0, 0)
    load_idx(1, 1)

    def zb(i, _):
        def zb2(j, _):
            rows_v[0, i, pl.ds(j * 16, 16)] = jnp.zeros((16,), jnp.float32)
            return 0
        lax.fori_loop(0, D // 16, zb2, 0)
        return 0

    lax.fori_loop(0, CH, zb, 0)

    def zloop(k, _):
        pltpu.sync_copy(rows_v.at[0], acc_sh.at[pl.ds(s * RPT + k * CH, CH)])
        return 0

    lax.fori_loop(0, RPT // CH, zloop, 0)
    plsc.subcore_barrier()

    def gather(q, u, buf):
        pltpu.async_copy(h2_hbm.at[ridx_v.at[q, u]], rows_v.at[buf],
                         semG.at[buf])

    def gather_wait(q, u, buf):
        pltpu.make_async_copy(h2_hbm.at[ridx_v.at[q, u]], rows_v.at[buf],
                              semG.at[buf]).wait()

    def scat(q, u, buf):
        pltpu.sync_copy(rows_v.at[buf], acc_sh.at[cidx_v.at[q, u]], add=True)

    def body(tp, _):
        for tt in range(2):
            t = tp * 2 + tt
            idx_wait(t, tt)
            gather(tt, 0, 0)
            gather(tt, 1, 1)
            for u in range(SUP):
                gather_wait(tt, u, u % 2)
                scat(tt, u, u % 2)
                if u + 2 < SUP:
                    gather(tt, u + 2, u % 2)

            @pl.when(tp < NSU // 2 - 1)
            def _():
                load_idx(t + 2, tt)
        return 0

    lax.fori_loop(0, NSU // 2, body, 0)

    plsc.subcore_barrier()

    @pl.when(c == 0)
    def _():
        pltpu.sync_copy(acc_sh.at[pl.ds(s * RPT, RPT)],
                        p0_hbm.at[pl.ds(s * RPT, RPT)])

    @pl.when(c == 1)
    def _():
        pltpu.sync_copy(acc_sh.at[pl.ds(s * RPT, RPT)],
                        p1_hbm.at[pl.ds(s * RPT, RPT)])


_scat_call = functools.partial(
    pl.kernel,
    out_type=(
        jax.ShapeDtypeStruct((NP, D), jnp.float32),
        jax.ShapeDtypeStruct((NP, D), jnp.float32),
    ),
    mesh=_MESH,
    scratch_types=[
        pltpu.VMEM((2, SUP, CH), jnp.int32),
        pltpu.VMEM((2, SUP, CH), jnp.int32),
        pltpu.VMEM((NBUF, CH, D), jnp.float32),
        pltpu.VMEM_SHARED((NP, D), jnp.float32),
        pltpu.SemaphoreType.DMA((2,)),
        pltpu.SemaphoreType.DMA((NBUF,)),
    ],
)(_scat_body)


def _comb_body(p0_ref, p1_ref, h2_ref, dinv_ref, b_ref, out_ref):
    out_ref[...] = (
        dinv_ref[...] * (p0_ref[...] + p1_ref[...] + h2_ref[...]) + b_ref[...]
    )


def _comb_call(p0, p1, h2, dinv, b2):
    grid = N // _MMR
    return pl.pallas_call(
        _comb_body,
        grid=(grid,),
        in_specs=[
            pl.BlockSpec((_MMR, D), lambda i: (i, 0)),
            pl.BlockSpec((_MMR, D), lambda i: (i, 0)),
            pl.BlockSpec((_MMR, D), lambda i: (i, 0)),
            pl.BlockSpec((_MMR, 1), lambda i: (i, 0)),
            pl.BlockSpec((1, D), lambda i: (0, 0)),
        ],
        out_specs=pl.BlockSpec((_MMR, D), lambda i: (i, 0)),
        out_shape=jax.ShapeDtypeStruct((N, D), jnp.float32),
    )(p0, p1, h2, dinv, b2)


def kernel(x, edge_index, W, b):
    row = edge_index[0]
    col = edge_index[1]
    pad = E_PAD - E
    pad_i = jnp.arange(pad, dtype=jnp.int32)
    row_p = jnp.concatenate([row, pad_i % N])
    col_p = jnp.concatenate([col, N + pad_i % (NP - N)])
    row3 = row_p.reshape(NW, CPT, CH)
    col3 = col_p.reshape(NW, CPT, CH)

    deg0, deg1 = _deg_call(col3)
    d0 = deg0[:N, None]
    d1 = deg1[:N, None]
    h2, dinv = _mm_call(x, W, d0, d1)
    p0, p1 = _scat_call(row3, col3, h2)
    b2 = b[None, :]
    return _comb_call(p0, p1, h2, dinv, b2)


# speedup vs baseline: 3.3924x; 1.0615x over previous
import functools

import jax
import jax.numpy as jnp
from jax import lax
from jax.experimental import pallas as pl
from jax.experimental.pallas import tpu as pltpu
from jax.experimental.pallas import tpu_sc as plsc

N = 10000
E = 320000
D = 128

NC, NS = 2, 16
NW = NC * NS
CH = 128
CPT = 80
E_PAD = CPT * CH * NW
NP = 10240
RPT = NP // NS
SUP = 8
NSU = CPT // SUP

_MESH = plsc.VectorSubcoreMesh(
    core_axis_name="c", subcore_axis_name="s", num_cores=NC, num_subcores=NS
)


def _worker():
    return lax.axis_index("s") * NC + lax.axis_index("c")


def _deg_body(col_hbm, deg0_hbm, deg1_hbm, ones_v, cidx_v, zrow_v, deg_sh, sem):
    c = lax.axis_index("c")
    s = lax.axis_index("s")
    w = _worker()

    def fill(i, _):
        ones_v[pl.ds(i * 16, 16)] = jnp.ones((16,), jnp.float32)
        zrow_v[pl.ds(i * 16, 16)] = jnp.zeros((16,), jnp.float32)
        return 0

    lax.fori_loop(0, CH // 16, fill, 0)

    pltpu.async_copy(col_hbm.at[w], cidx_v, sem).wait()

    def zloop(i, _):
        pltpu.sync_copy(zrow_v, deg_sh.at[pl.ds(s * RPT + i * CH, CH)])
        return 0

    lax.fori_loop(0, RPT // CH, zloop, 0)
    plsc.subcore_barrier()

    def body(j, _):
        pltpu.sync_copy(ones_v, deg_sh.at[cidx_v.at[j]], add=True)
        return 0

    lax.fori_loop(0, CPT, body, 0)
    plsc.subcore_barrier()

    @pl.when(c == 0)
    def _():
        pltpu.sync_copy(deg_sh.at[pl.ds(s * RPT, RPT)],
                        deg0_hbm.at[pl.ds(s * RPT, RPT)])

    @pl.when(c == 1)
    def _():
        pltpu.sync_copy(deg_sh.at[pl.ds(s * RPT, RPT)],
                        deg1_hbm.at[pl.ds(s * RPT, RPT)])


_deg_call = functools.partial(
    pl.kernel,
    out_type=(
        jax.ShapeDtypeStruct((NP,), jnp.float32),
        jax.ShapeDtypeStruct((NP,), jnp.float32),
    ),
    mesh=_MESH,
    scratch_types=[
        pltpu.VMEM((CH,), jnp.float32),
        pltpu.VMEM((CPT, CH), jnp.int32),
        pltpu.VMEM((CH,), jnp.float32),
        pltpu.VMEM_SHARED((NP,), jnp.float32),
        pltpu.SemaphoreType.DMA,
    ],
)(_deg_body)


def _mm_body(x_ref, w_ref, d0_ref, d1_ref, h2_ref, dinv_ref):
    deg = d0_ref[...] + d1_ref[...] + 1.0
    dinv = lax.rsqrt(deg)
    h = jnp.dot(x_ref[...], w_ref[...], preferred_element_type=jnp.float32)
    h2_ref[...] = h * dinv
    dinv_ref[...] = dinv


_MMR = 2000


def _mm_call(x, W, d0, d1):
    grid = N // _MMR
    return pl.pallas_call(
        _mm_body,
        grid=(grid,),
        in_specs=[
            pl.BlockSpec((_MMR, D), lambda i: (i, 0)),
            pl.BlockSpec((D, D), lambda i: (0, 0)),
            pl.BlockSpec((_MMR, 1), lambda i: (i, 0)),
            pl.BlockSpec((_MMR, 1), lambda i: (i, 0)),
        ],
        out_specs=[
            pl.BlockSpec((_MMR, D), lambda i: (i, 0)),
            pl.BlockSpec((_MMR, 1), lambda i: (i, 0)),
        ],
        out_shape=[
            jax.ShapeDtypeStruct((N, D), jnp.float32),
            jax.ShapeDtypeStruct((N, 1), jnp.float32),
        ],
    )(x, W, d0, d1)


def _scat_body(row_hbm, col_hbm, h2_hbm, p0_hbm, p1_hbm,
               ridx_v, cidx_v, rows_v, acc_sh, semI, semG, semS):
    c = lax.axis_index("c")
    s = lax.axis_index("s")
    w = _worker()

    def load_idx(t, q):
        src_r = row_hbm.at[w, pl.ds(pl.multiple_of(t * SUP, SUP), SUP)]
        src_c = col_hbm.at[w, pl.ds(pl.multiple_of(t * SUP, SUP), SUP)]
        pltpu.async_copy(src_r, ridx_v.at[q], semI.at[q])
        pltpu.async_copy(src_c, cidx_v.at[q], semI.at[q])

    def idx_wait(t, q):
        src_r = row_hbm.at[w, pl.ds(pl.multiple_of(t * SUP, SUP), SUP)]
        src_c = col_hbm.at[w, pl.ds(pl.multiple_of(t * SUP, SUP), SUP)]
        pltpu.make_async_copy(src_r, ridx_v.at[q], semI.at[q]).wait()
        pltpu.make_async_copy(src_c, cidx_v.at[q], semI.at[q]).wait()

    load_idx(0, 0)
    load_idx(1, 1)

    def zb(i, _):
        def zb2(j, _):
            rows_v[0, i, pl.ds(j * 16, 16)] = jnp.zeros((16,), jnp.float32)
            return 0
        lax.fori_loop(0, D // 16, zb2, 0)
        return 0

    lax.fori_loop(0, CH, zb, 0)

    for k in range(RPT // CH):
        pltpu.sync_copy(rows_v.at[0], acc_sh.at[pl.ds(s * RPT + k * CH, CH)])
    plsc.subcore_barrier()

    def gather(q, u, buf):
        pltpu.async_copy(h2_hbm.at[ridx_v.at[q, u]], rows_v.at[buf],
                         semG.at[buf])

    def gather_wait(q, u, buf):
        pltpu.make_async_copy(h2_hbm.at[ridx_v.at[q, u]], rows_v.at[buf],
                              semG.at[buf]).wait()

    def scat_start(q, u, buf):
        pltpu.async_copy(rows_v.at[buf], acc_sh.at[cidx_v.at[q, u]],
                         semS.at[buf], add=True)

    def scat_wait(q, u, buf):
        pltpu.make_async_copy(rows_v.at[buf], acc_sh.at[cidx_v.at[q, u]],
                              semS.at[buf]).wait()

    idx_wait(0, 0)
    gather(0, 0, 0)

    def body(tp, _):
        for tt in range(2):
            t = tp * 2 + tt
            for u in range(SUP):
                j = t * SUP + u
                buf = u % 2
                qm = (tt ^ 1, SUP - 1) if u == 0 else (tt, u - 1)

                @pl.when(j > 0)
                def _():
                    scat_wait(qm[0], qm[1], buf ^ 1)

                if u == 0:
                    @pl.when(jnp.logical_and(t >= 1, t + 1 < NSU))
                    def _():
                        load_idx(t + 1, tt ^ 1)

                if u == SUP - 1:
                    @pl.when(t + 1 < NSU)
                    def _():
                        idx_wait(t + 1, tt ^ 1)
                        gather(tt ^ 1, 0, buf ^ 1)
                else:
                    gather(tt, u + 1, buf ^ 1)

                gather_wait(tt, u, buf)
                scat_start(tt, u, buf)
        return 0

    lax.fori_loop(0, NSU // 2, body, 0)
    scat_wait(1, SUP - 1, 1)
    plsc.subcore_barrier()

    @pl.when(c == 0)
    def _():
        pltpu.sync_copy(acc_sh.at[pl.ds(s * RPT, RPT)],
                        p0_hbm.at[pl.ds(s * RPT, RPT)])

    @pl.when(c == 1)
    def _():
        pltpu.sync_copy(acc_sh.at[pl.ds(s * RPT, RPT)],
                        p1_hbm.at[pl.ds(s * RPT, RPT)])


_scat_call = functools.partial(
    pl.kernel,
    out_type=(
        jax.ShapeDtypeStruct((NP, D), jnp.float32),
        jax.ShapeDtypeStruct((NP, D), jnp.float32),
    ),
    mesh=_MESH,
    scratch_types=[
        pltpu.VMEM((2, SUP, CH), jnp.int32),
        pltpu.VMEM((2, SUP, CH), jnp.int32),
        pltpu.VMEM((2, CH, D), jnp.float32),
        pltpu.VMEM_SHARED((NP, D), jnp.float32),
        pltpu.SemaphoreType.DMA((2,)),
        pltpu.SemaphoreType.DMA((2,)),
        pltpu.SemaphoreType.DMA((2,)),
    ],
)(_scat_body)


def _comb_body(p0_ref, p1_ref, h2_ref, dinv_ref, b_ref, out_ref):
    out_ref[...] = (
        dinv_ref[...] * (p0_ref[...] + p1_ref[...] + h2_ref[...]) + b_ref[...]
    )


def _comb_call(p0, p1, h2, dinv, b2):
    grid = N // _MMR
    return pl.pallas_call(
        _comb_body,
        grid=(grid,),
        in_specs=[
            pl.BlockSpec((_MMR, D), lambda i: (i, 0)),
            pl.BlockSpec((_MMR, D), lambda i: (i, 0)),
            pl.BlockSpec((_MMR, D), lambda i: (i, 0)),
            pl.BlockSpec((_MMR, 1), lambda i: (i, 0)),
            pl.BlockSpec((1, D), lambda i: (0, 0)),
        ],
        out_specs=pl.BlockSpec((_MMR, D), lambda i: (i, 0)),
        out_shape=jax.ShapeDtypeStruct((N, D), jnp.float32),
    )(p0, p1, h2, dinv, b2)


def kernel(x, edge_index, W, b):
    row = edge_index[0]
    col = edge_index[1]
    pad = E_PAD - E
    pad_i = jnp.arange(pad, dtype=jnp.int32)
    row_p = jnp.concatenate([row, pad_i % N])
    col_p = jnp.concatenate([col, N + pad_i % (NP - N)])
    row3 = row_p.reshape(NW, CPT, CH)
    col3 = col_p.reshape(NW, CPT, CH)

    deg0, deg1 = _deg_call(col3)
    d0 = deg0[:N, None]
    d1 = deg1[:N, None]
    h2, dinv = _mm_call(x, W, d0, d1)
    p0, p1 = _scat_call(row3, col3, h2)
    b2 = b[None, :]
    return _comb_call(p0, p1, h2, dinv, b2)


# speedup vs baseline: 3.4833x; 1.0268x over previous
import functools

import jax
import jax.numpy as jnp
from jax import lax
from jax.experimental import pallas as pl
from jax.experimental.pallas import tpu as pltpu
from jax.experimental.pallas import tpu_sc as plsc

N = 10000
E = 320000
D = 128

NC, NS = 2, 16
NW = NC * NS
CH = 128
CPT = 80
E_PAD = CPT * CH * NW
NP = 10240
RPT = NP // NS
SUP = 8
NSU = CPT // SUP

_MESH = plsc.VectorSubcoreMesh(
    core_axis_name="c", subcore_axis_name="s", num_cores=NC, num_subcores=NS
)


def _worker():
    return lax.axis_index("s") * NC + lax.axis_index("c")


def _deg_body(col_hbm, deg0_hbm, deg1_hbm, ones_v, cidx_v, zrow_v, deg_sh, sem):
    c = lax.axis_index("c")
    s = lax.axis_index("s")
    w = _worker()

    def fill(i, _):
        ones_v[pl.ds(i * 16, 16)] = jnp.ones((16,), jnp.float32)
        zrow_v[pl.ds(i * 16, 16)] = jnp.zeros((16,), jnp.float32)
        return 0

    lax.fori_loop(0, CH // 16, fill, 0)

    pltpu.async_copy(col_hbm.at[w], cidx_v, sem).wait()

    def zloop(i, _):
        pltpu.sync_copy(zrow_v, deg_sh.at[pl.ds(s * RPT + i * CH, CH)])
        return 0

    lax.fori_loop(0, RPT // CH, zloop, 0)
    plsc.subcore_barrier()

    def body(j, _):
        pltpu.async_copy(ones_v, deg_sh.at[cidx_v.at[j]], sem, add=True)
        return 0

    lax.fori_loop(0, CPT, body, 0)

    def drain(j, _):
        pltpu.make_async_copy(ones_v, deg_sh.at[cidx_v.at[j]], sem).wait()
        return 0

    lax.fori_loop(0, CPT, drain, 0)
    plsc.subcore_barrier()

    @pl.when(c == 0)
    def _():
        pltpu.sync_copy(deg_sh.at[pl.ds(s * RPT, RPT)],
                        deg0_hbm.at[pl.ds(s * RPT, RPT)])

    @pl.when(c == 1)
    def _():
        pltpu.sync_copy(deg_sh.at[pl.ds(s * RPT, RPT)],
                        deg1_hbm.at[pl.ds(s * RPT, RPT)])


_deg_call = functools.partial(
    pl.kernel,
    out_type=(
        jax.ShapeDtypeStruct((NP,), jnp.float32),
        jax.ShapeDtypeStruct((NP,), jnp.float32),
    ),
    mesh=_MESH,
    scratch_types=[
        pltpu.VMEM((CH,), jnp.float32),
        pltpu.VMEM((CPT, CH), jnp.int32),
        pltpu.VMEM((CH,), jnp.float32),
        pltpu.VMEM_SHARED((NP,), jnp.float32),
        pltpu.SemaphoreType.DMA,
    ],
)(_deg_body)


def _mm_body(x_ref, w_ref, d0_ref, d1_ref, h2_ref, dinv_ref):
    deg = d0_ref[...] + d1_ref[...] + 1.0
    dinv = lax.rsqrt(deg)
    h = jnp.dot(x_ref[...], w_ref[...], preferred_element_type=jnp.float32)
    h2_ref[...] = h * dinv
    dinv_ref[...] = dinv


_MMR = 2000


def _mm_call(x, W, d0, d1):
    grid = N // _MMR
    return pl.pallas_call(
        _mm_body,
        grid=(grid,),
        in_specs=[
            pl.BlockSpec((_MMR, D), lambda i: (i, 0)),
            pl.BlockSpec((D, D), lambda i: (0, 0)),
            pl.BlockSpec((_MMR, 1), lambda i: (i, 0)),
            pl.BlockSpec((_MMR, 1), lambda i: (i, 0)),
        ],
        out_specs=[
            pl.BlockSpec((_MMR, D), lambda i: (i, 0)),
            pl.BlockSpec((_MMR, 1), lambda i: (i, 0)),
        ],
        out_shape=[
            jax.ShapeDtypeStruct((N, D), jnp.float32),
            jax.ShapeDtypeStruct((N, 1), jnp.float32),
        ],
    )(x, W, d0, d1)


def _scat_body(row_hbm, col_hbm, h2_hbm, p0_hbm, p1_hbm,
               ridx_v, cidx_v, rows_v, acc_sh, semI, semG, semS):
    c = lax.axis_index("c")
    s = lax.axis_index("s")
    w = _worker()

    def load_idx(t, q):
        src_r = row_hbm.at[w, pl.ds(pl.multiple_of(t * SUP, SUP), SUP)]
        src_c = col_hbm.at[w, pl.ds(pl.multiple_of(t * SUP, SUP), SUP)]
        pltpu.async_copy(src_r, ridx_v.at[q], semI.at[q])
        pltpu.async_copy(src_c, cidx_v.at[q], semI.at[q])

    def idx_wait(t, q):
        src_r = row_hbm.at[w, pl.ds(pl.multiple_of(t * SUP, SUP), SUP)]
        src_c = col_hbm.at[w, pl.ds(pl.multiple_of(t * SUP, SUP), SUP)]
        pltpu.make_async_copy(src_r, ridx_v.at[q], semI.at[q]).wait()
        pltpu.make_async_copy(src_c, cidx_v.at[q], semI.at[q]).wait()

    load_idx(0, 0)
    load_idx(1, 1)

    def zb(i, _):
        def zb2(j, _):
            rows_v[0, i, pl.ds(j * 16, 16)] = jnp.zeros((16,), jnp.float32)
            return 0
        lax.fori_loop(0, D // 16, zb2, 0)
        return 0

    lax.fori_loop(0, CH, zb, 0)

    for k in range(RPT // CH):
        pltpu.sync_copy(rows_v.at[0], acc_sh.at[pl.ds(s * RPT + k * CH, CH)])
    plsc.subcore_barrier()

    def gather(q, u, buf):
        pltpu.async_copy(h2_hbm.at[ridx_v.at[q, u]], rows_v.at[buf],
                         semG.at[buf])

    def gather_wait(q, u, buf):
        pltpu.make_async_copy(h2_hbm.at[ridx_v.at[q, u]], rows_v.at[buf],
                              semG.at[buf]).wait()

    def scat_start(q, u, buf):
        pltpu.async_copy(rows_v.at[buf], acc_sh.at[cidx_v.at[q, u]],
                         semS.at[buf], add=True)

    def scat_wait(q, u, buf):
        pltpu.make_async_copy(rows_v.at[buf], acc_sh.at[cidx_v.at[q, u]],
                              semS.at[buf]).wait()

    idx_wait(0, 0)
    gather(0, 0, 0)

    def body(tp, _):
        for tt in range(2):
            t = tp * 2 + tt
            for u in range(SUP):
                j = t * SUP + u
                buf = u % 2
                qm = (tt ^ 1, SUP - 1) if u == 0 else (tt, u - 1)

                @pl.when(j > 0)
                def _():
                    scat_wait(qm[0], qm[1], buf ^ 1)

                if u == 0:
                    @pl.when(jnp.logical_and(t >= 1, t + 1 < NSU))
                    def _():
                        load_idx(t + 1, tt ^ 1)

                if u == SUP - 1:
                    @pl.when(t + 1 < NSU)
                    def _():
                        idx_wait(t + 1, tt ^ 1)
                        gather(tt ^ 1, 0, buf ^ 1)
                else:
                    gather(tt, u + 1, buf ^ 1)

                gather_wait(tt, u, buf)
                scat_start(tt, u, buf)
        return 0

    lax.fori_loop(0, NSU // 2, body, 0)
    scat_wait(1, SUP - 1, 1)
    plsc.subcore_barrier()

    @pl.when(c == 0)
    def _():
        pltpu.sync_copy(acc_sh.at[pl.ds(s * RPT, RPT)],
                        p0_hbm.at[pl.ds(s * RPT, RPT)])

    @pl.when(c == 1)
    def _():
        pltpu.sync_copy(acc_sh.at[pl.ds(s * RPT, RPT)],
                        p1_hbm.at[pl.ds(s * RPT, RPT)])


_scat_call = functools.partial(
    pl.kernel,
    out_type=(
        jax.ShapeDtypeStruct((NP, D), jnp.float32),
        jax.ShapeDtypeStruct((NP, D), jnp.float32),
    ),
    mesh=_MESH,
    scratch_types=[
        pltpu.VMEM((2, SUP, CH), jnp.int32),
        pltpu.VMEM((2, SUP, CH), jnp.int32),
        pltpu.VMEM((2, CH, D), jnp.float32),
        pltpu.VMEM_SHARED((NP, D), jnp.float32),
        pltpu.SemaphoreType.DMA((2,)),
        pltpu.SemaphoreType.DMA((2,)),
        pltpu.SemaphoreType.DMA((2,)),
    ],
)(_scat_body)


def _comb_body(p0_ref, p1_ref, h2_ref, dinv_ref, b_ref, out_ref):
    out_ref[...] = (
        dinv_ref[...] * (p0_ref[...] + p1_ref[...] + h2_ref[...]) + b_ref[...]
    )


def _comb_call(p0, p1, h2, dinv, b2):
    grid = N // _MMR
    return pl.pallas_call(
        _comb_body,
        grid=(grid,),
        in_specs=[
            pl.BlockSpec((_MMR, D), lambda i: (i, 0)),
            pl.BlockSpec((_MMR, D), lambda i: (i, 0)),
            pl.BlockSpec((_MMR, D), lambda i: (i, 0)),
            pl.BlockSpec((_MMR, 1), lambda i: (i, 0)),
            pl.BlockSpec((1, D), lambda i: (0, 0)),
        ],
        out_specs=pl.BlockSpec((_MMR, D), lambda i: (i, 0)),
        out_shape=jax.ShapeDtypeStruct((N, D), jnp.float32),
    )(p0, p1, h2, dinv, b2)


def kernel(x, edge_index, W, b):
    row = edge_index[0]
    col = edge_index[1]
    pad = E_PAD - E
    pad_i = jnp.arange(pad, dtype=jnp.int32)
    row_p = jnp.concatenate([row, pad_i % N])
    col_p = jnp.concatenate([col, N + pad_i % (NP - N)])
    row3 = row_p.reshape(NW, CPT, CH)
    col3 = col_p.reshape(NW, CPT, CH)

    deg0, deg1 = _deg_call(col3)
    d0 = deg0[:N, None]
    d1 = deg1[:N, None]
    h2, dinv = _mm_call(x, W, d0, d1)
    p0, p1 = _scat_call(row3, col3, h2)
    b2 = b[None, :]
    return _comb_call(p0, p1, h2, dinv, b2)


# speedup vs baseline: 3.5639x; 1.0232x over previous
import functools

import jax
import jax.numpy as jnp
from jax import lax
from jax.experimental import pallas as pl
from jax.experimental.pallas import tpu as pltpu
from jax.experimental.pallas import tpu_sc as plsc

N = 10000
E = 320000
D = 128

NC, NS = 2, 16
NW = NC * NS
CH = 128
CPT = 80
E_PAD = CPT * CH * NW
NP = 10240
RPT = NP // NS
SUP = 8
NSU = CPT // SUP

_MESH = plsc.VectorSubcoreMesh(
    core_axis_name="c", subcore_axis_name="s", num_cores=NC, num_subcores=NS
)


def _worker():
    return lax.axis_index("s") * NC + lax.axis_index("c")


def _deg_body(col_hbm, deg0_hbm, deg1_hbm, ones_v, cidx_v, zrow_v, deg_sh, sem):
    c = lax.axis_index("c")
    s = lax.axis_index("s")
    w = _worker()

    def fill(i, _):
        ones_v[pl.ds(i * 16, 16)] = jnp.ones((16,), jnp.float32)
        zrow_v[pl.ds(i * 16, 16)] = jnp.zeros((16,), jnp.float32)
        return 0

    lax.fori_loop(0, CH // 16, fill, 0)

    pltpu.async_copy(col_hbm.at[w], cidx_v, sem).wait()

    def zloop(i, _):
        pltpu.sync_copy(zrow_v, deg_sh.at[pl.ds(s * RPT + i * CH, CH)])
        return 0

    lax.fori_loop(0, RPT // CH, zloop, 0)
    plsc.subcore_barrier()

    def body(j, _):
        pltpu.async_copy(ones_v, deg_sh.at[cidx_v.at[j]], sem, add=True)
        return 0

    lax.fori_loop(0, CPT, body, 0)

    def drain(j, _):
        pltpu.make_async_copy(ones_v, deg_sh.at[cidx_v.at[j]], sem).wait()
        return 0

    lax.fori_loop(0, CPT, drain, 0)
    plsc.subcore_barrier()

    @pl.when(c == 0)
    def _():
        pltpu.sync_copy(deg_sh.at[pl.ds(s * RPT, RPT)],
                        deg0_hbm.at[pl.ds(s * RPT, RPT)])

    @pl.when(c == 1)
    def _():
        pltpu.sync_copy(deg_sh.at[pl.ds(s * RPT, RPT)],
                        deg1_hbm.at[pl.ds(s * RPT, RPT)])


_deg_call = functools.partial(
    pl.kernel,
    out_type=(
        jax.ShapeDtypeStruct((NP,), jnp.float32),
        jax.ShapeDtypeStruct((NP,), jnp.float32),
    ),
    mesh=_MESH,
    scratch_types=[
        pltpu.VMEM((CH,), jnp.float32),
        pltpu.VMEM((CPT, CH), jnp.int32),
        pltpu.VMEM((CH,), jnp.float32),
        pltpu.VMEM_SHARED((NP,), jnp.float32),
        pltpu.SemaphoreType.DMA,
    ],
)(_deg_body)


def _mm_body(x_ref, w_ref, d0_ref, d1_ref, h2_ref, dinv_ref):
    deg = d0_ref[...] + d1_ref[...] + 1.0
    dinv = lax.rsqrt(deg)
    h = jnp.dot(x_ref[...], w_ref[...], preferred_element_type=jnp.float32)
    h2_ref[...] = h * dinv
    dinv_ref[...] = dinv


_MMR = 2000


def _mm_call(x, W, d0, d1):
    grid = N // _MMR
    return pl.pallas_call(
        _mm_body,
        grid=(grid,),
        in_specs=[
            pl.BlockSpec((_MMR, D), lambda i: (i, 0)),
            pl.BlockSpec((D, D), lambda i: (0, 0)),
            pl.BlockSpec((_MMR, 1), lambda i: (i, 0)),
            pl.BlockSpec((_MMR, 1), lambda i: (i, 0)),
        ],
        out_specs=[
            pl.BlockSpec((_MMR, D), lambda i: (i, 0)),
            pl.BlockSpec((_MMR, 1), lambda i: (i, 0)),
        ],
        out_shape=[
            jax.ShapeDtypeStruct((N, D), jnp.float32),
            jax.ShapeDtypeStruct((N, 1), jnp.float32),
        ],
    )(x, W, d0, d1)


def _scat_body(row_hbm, col_hbm, h2_hbm, p0_hbm, p1_hbm,
               ridx_v, cidx_v, rows_v, acc_sh, semI, semG, semS):
    c = lax.axis_index("c")
    s = lax.axis_index("s")
    w = _worker()

    def load_idx(t, q):
        src_r = row_hbm.at[w, pl.ds(pl.multiple_of(t * SUP, SUP), SUP)]
        src_c = col_hbm.at[w, pl.ds(pl.multiple_of(t * SUP, SUP), SUP)]
        pltpu.async_copy(src_r, ridx_v.at[q], semI.at[q])
        pltpu.async_copy(src_c, cidx_v.at[q], semI.at[q])

    def idx_wait(t, q):
        src_r = row_hbm.at[w, pl.ds(pl.multiple_of(t * SUP, SUP), SUP)]
        src_c = col_hbm.at[w, pl.ds(pl.multiple_of(t * SUP, SUP), SUP)]
        pltpu.make_async_copy(src_r, ridx_v.at[q], semI.at[q]).wait()
        pltpu.make_async_copy(src_c, cidx_v.at[q], semI.at[q]).wait()

    load_idx(0, 0)
    load_idx(1, 1)

    def zb(i, _):
        def zb2(j, _):
            rows_v[0, i, pl.ds(j * 16, 16)] = jnp.zeros((16,), jnp.float32)
            return 0
        lax.fori_loop(0, D // 16, zb2, 0)
        return 0

    lax.fori_loop(0, CH, zb, 0)

    for k in range(RPT // CH):
        pltpu.sync_copy(rows_v.at[0], acc_sh.at[pl.ds(s * RPT + k * CH, CH)])
    plsc.subcore_barrier()

    def gather(q, u, buf):
        pltpu.async_copy(h2_hbm.at[ridx_v.at[q, u]], rows_v.at[buf],
                         semG.at[buf])

    def gather_wait(q, u, buf):
        pltpu.make_async_copy(h2_hbm.at[ridx_v.at[q, u]], rows_v.at[buf],
                              semG.at[buf]).wait()

    def scat_start(q, u, buf):
        pltpu.async_copy(rows_v.at[buf], acc_sh.at[cidx_v.at[q, u]],
                         semS.at[buf], add=True)

    def scat_wait(q, u, buf):
        pltpu.make_async_copy(rows_v.at[buf], acc_sh.at[cidx_v.at[q, u]],
                              semS.at[buf]).wait()

    idx_wait(0, 0)
    gather(0, 0, 0)

    def body(tp, _):
        for tt in range(2):
            t = tp * 2 + tt
            for u in range(SUP):
                j = t * SUP + u
                buf = u % 2
                qm = (tt ^ 1, SUP - 1) if u == 0 else (tt, u - 1)

                @pl.when(j > 0)
                def _():
                    scat_wait(qm[0], qm[1], buf ^ 1)

                if u == 0:
                    @pl.when(jnp.logical_and(t >= 1, t + 1 < NSU))
                    def _():
                        load_idx(t + 1, tt ^ 1)

                if u == SUP - 1:
                    @pl.when(t + 1 < NSU)
                    def _():
                        idx_wait(t + 1, tt ^ 1)
                        gather(tt ^ 1, 0, buf ^ 1)
                else:
                    gather(tt, u + 1, buf ^ 1)

                gather_wait(tt, u, buf)
                scat_start(tt, u, buf)
        return 0

    lax.fori_loop(0, NSU // 2, body, 0)
    scat_wait(1, SUP - 1, 1)
    plsc.subcore_barrier()

    @pl.when(c == 0)
    def _():
        pltpu.sync_copy(acc_sh.at[pl.ds(s * RPT, RPT)],
                        p0_hbm.at[pl.ds(s * RPT, RPT)])

    @pl.when(c == 1)
    def _():
        pltpu.sync_copy(acc_sh.at[pl.ds(s * RPT, RPT)],
                        p1_hbm.at[pl.ds(s * RPT, RPT)])


_scat_call = functools.partial(
    pl.kernel,
    out_type=(
        jax.ShapeDtypeStruct((NP, D), jnp.float32),
        jax.ShapeDtypeStruct((NP, D), jnp.float32),
    ),
    mesh=_MESH,
    scratch_types=[
        pltpu.VMEM((2, SUP, CH), jnp.int32),
        pltpu.VMEM((2, SUP, CH), jnp.int32),
        pltpu.VMEM((2, CH, D), jnp.float32),
        pltpu.VMEM_SHARED((NP, D), jnp.float32),
        pltpu.SemaphoreType.DMA((2,)),
        pltpu.SemaphoreType.DMA((2,)),
        pltpu.SemaphoreType.DMA((2,)),
    ],
)(_scat_body)


def _comb_body(p0_ref, p1_ref, h2_ref, dinv_ref, b_ref, out_ref):
    out_ref[...] = (
        dinv_ref[...] * (p0_ref[...] + p1_ref[...] + h2_ref[...]) + b_ref[...]
    )


def _comb_call(p0, p1, h2, dinv, b2):
    grid = N // _MMR
    return pl.pallas_call(
        _comb_body,
        grid=(grid,),
        in_specs=[
            pl.BlockSpec((_MMR, D), lambda i: (i, 0)),
            pl.BlockSpec((_MMR, D), lambda i: (i, 0)),
            pl.BlockSpec((_MMR, D), lambda i: (i, 0)),
            pl.BlockSpec((_MMR, 1), lambda i: (i, 0)),
            pl.BlockSpec((1, D), lambda i: (0, 0)),
        ],
        out_specs=pl.BlockSpec((_MMR, D), lambda i: (i, 0)),
        out_shape=jax.ShapeDtypeStruct((N, D), jnp.float32),
    )(p0, p1, h2, dinv, b2)


def kernel(x, edge_index, W, b):
    row = edge_index[0]
    col = edge_index[1]
    pad = E_PAD - E
    pad_i = jnp.arange(pad, dtype=jnp.int32)
    row_p = jnp.concatenate([row, pad_i % N])
    col_p = jnp.concatenate([col, N + pad_i % (NP - N)])
    row3 = row_p.reshape(NW, CPT, CH)
    col3 = col_p.reshape(NW, CPT, CH)

    deg0, deg1 = _deg_call(col3)
    d0 = deg0.reshape(NP, 1)
    d1 = deg1.reshape(NP, 1)
    h2, dinv = _mm_call(x, W, d0, d1)
    p0, p1 = _scat_call(row3, col3, h2)
    b2 = b[None, :]
    return _comb_call(p0, p1, h2, dinv, b2)


# speedup vs baseline: 3.5918x; 1.0078x over previous
import functools

import jax
import jax.numpy as jnp
import numpy as np
from jax import lax
from jax.experimental import pallas as pl
from jax.experimental.pallas import tpu as pltpu
from jax.experimental.pallas import tpu_sc as plsc

N = 10000
E = 320000
D = 128

NC, NS = 2, 16
NW = NC * NS
CH = 128
CPT = 80
E_PAD = CPT * CH * NW
NP = 10240
RPT = NP // NS
SUP = 8
NSU = CPT // SUP

_PAD_I = np.arange(E_PAD - E, dtype=np.int32)
_ROW_PAD = jnp.asarray(_PAD_I % N)
_COL_PAD = jnp.asarray(N + _PAD_I % (NP - N))

_MESH = plsc.VectorSubcoreMesh(
    core_axis_name="c", subcore_axis_name="s", num_cores=NC, num_subcores=NS
)


def _worker():
    return lax.axis_index("s") * NC + lax.axis_index("c")


def _deg_body(col_hbm, deg0_hbm, deg1_hbm, ones_v, cidx_v, zrow_v, deg_sh, sem):
    c = lax.axis_index("c")
    s = lax.axis_index("s")
    w = _worker()

    def fill(i, _):
        ones_v[pl.ds(i * 16, 16)] = jnp.ones((16,), jnp.float32)
        zrow_v[pl.ds(i * 16, 16)] = jnp.zeros((16,), jnp.float32)
        return 0

    lax.fori_loop(0, CH // 16, fill, 0)

    pltpu.async_copy(col_hbm.at[w], cidx_v, sem).wait()

    def zloop(i, _):
        pltpu.sync_copy(zrow_v, deg_sh.at[pl.ds(s * RPT + i * CH, CH)])
        return 0

    lax.fori_loop(0, RPT // CH, zloop, 0)
    plsc.subcore_barrier()

    def body(j, _):
        pltpu.async_copy(ones_v, deg_sh.at[cidx_v.at[j]], sem, add=True)
        return 0

    lax.fori_loop(0, CPT, body, 0)

    def drain(j, _):
        pltpu.make_async_copy(ones_v, deg_sh.at[cidx_v.at[j]], sem).wait()
        return 0

    lax.fori_loop(0, CPT, drain, 0)
    plsc.subcore_barrier()

    @pl.when(c == 0)
    def _():
        pltpu.sync_copy(deg_sh.at[pl.ds(s * RPT, RPT)],
                        deg0_hbm.at[pl.ds(s * RPT, RPT)])

    @pl.when(c == 1)
    def _():
        pltpu.sync_copy(deg_sh.at[pl.ds(s * RPT, RPT)],
                        deg1_hbm.at[pl.ds(s * RPT, RPT)])


_deg_call = functools.partial(
    pl.kernel,
    out_type=(
        jax.ShapeDtypeStruct((NP,), jnp.float32),
        jax.ShapeDtypeStruct((NP,), jnp.float32),
    ),
    mesh=_MESH,
    scratch_types=[
        pltpu.VMEM((CH,), jnp.float32),
        pltpu.VMEM((CPT, CH), jnp.int32),
        pltpu.VMEM((CH,), jnp.float32),
        pltpu.VMEM_SHARED((NP,), jnp.float32),
        pltpu.SemaphoreType.DMA,
    ],
)(_deg_body)


def _mm_body(x_ref, w_ref, d0_ref, d1_ref, h2_ref, dinv_ref):
    deg = d0_ref[...] + d1_ref[...] + 1.0
    dinv = lax.rsqrt(deg)
    h = jnp.dot(x_ref[...], w_ref[...], preferred_element_type=jnp.float32)
    h2_ref[...] = h * dinv
    dinv_ref[...] = dinv


_MMR = 2000


def _mm_call(x, W, d0, d1):
    grid = N // _MMR
    return pl.pallas_call(
        _mm_body,
        grid=(grid,),
        in_specs=[
            pl.BlockSpec((_MMR, D), lambda i: (i, 0)),
            pl.BlockSpec((D, D), lambda i: (0, 0)),
            pl.BlockSpec((_MMR, 1), lambda i: (i, 0)),
            pl.BlockSpec((_MMR, 1), lambda i: (i, 0)),
        ],
        out_specs=[
            pl.BlockSpec((_MMR, D), lambda i: (i, 0)),
            pl.BlockSpec((_MMR, 1), lambda i: (i, 0)),
        ],
        out_shape=[
            jax.ShapeDtypeStruct((N, D), jnp.float32),
            jax.ShapeDtypeStruct((N, 1), jnp.float32),
        ],
    )(x, W, d0, d1)


def _scat_body(row_hbm, col_hbm, h2_hbm, p0_hbm, p1_hbm,
               ridx_v, cidx_v, rows_v, acc_sh, semI, semG, semS):
    c = lax.axis_index("c")
    s = lax.axis_index("s")
    w = _worker()

    def load_idx(t, q):
        src_r = row_hbm.at[w, pl.ds(pl.multiple_of(t * SUP, SUP), SUP)]
        src_c = col_hbm.at[w, pl.ds(pl.multiple_of(t * SUP, SUP), SUP)]
        pltpu.async_copy(src_r, ridx_v.at[q], semI.at[q])
        pltpu.async_copy(src_c, cidx_v.at[q], semI.at[q])

    def idx_wait(t, q):
        src_r = row_hbm.at[w, pl.ds(pl.multiple_of(t * SUP, SUP), SUP)]
        src_c = col_hbm.at[w, pl.ds(pl.multiple_of(t * SUP, SUP), SUP)]
        pltpu.make_async_copy(src_r, ridx_v.at[q], semI.at[q]).wait()
        pltpu.make_async_copy(src_c, cidx_v.at[q], semI.at[q]).wait()

    load_idx(0, 0)
    load_idx(1, 1)

    def zb(i, _):
        def zb2(j, _):
            rows_v[0, i, pl.ds(j * 16, 16)] = jnp.zeros((16,), jnp.float32)
            return 0
        lax.fori_loop(0, D // 16, zb2, 0)
        return 0

    lax.fori_loop(0, CH, zb, 0)

    for k in range(RPT // CH):
        pltpu.sync_copy(rows_v.at[0], acc_sh.at[pl.ds(s * RPT + k * CH, CH)])
    plsc.subcore_barrier()

    def gather(q, u, buf):
        pltpu.async_copy(h2_hbm.at[ridx_v.at[q, u]], rows_v.at[buf],
                         semG.at[buf])

    def gather_wait(q, u, buf):
        pltpu.make_async_copy(h2_hbm.at[ridx_v.at[q, u]], rows_v.at[buf],
                              semG.at[buf]).wait()

    def scat_start(q, u, buf):
        pltpu.async_copy(rows_v.at[buf], acc_sh.at[cidx_v.at[q, u]],
                         semS.at[buf], add=True)

    def scat_wait(q, u, buf):
        pltpu.make_async_copy(rows_v.at[buf], acc_sh.at[cidx_v.at[q, u]],
                              semS.at[buf]).wait()

    idx_wait(0, 0)
    gather(0, 0, 0)

    def body(tp, _):
        for tt in range(2):
            t = tp * 2 + tt
            for u in range(SUP):
                j = t * SUP + u
                buf = u % 2
                qm = (tt ^ 1, SUP - 1) if u == 0 else (tt, u - 1)

                @pl.when(j > 0)
                def _():
                    scat_wait(qm[0], qm[1], buf ^ 1)

                if u == 0:
                    @pl.when(jnp.logical_and(t >= 1, t + 1 < NSU))
                    def _():
                        load_idx(t + 1, tt ^ 1)

                if u == SUP - 1:
                    @pl.when(t + 1 < NSU)
                    def _():
                        idx_wait(t + 1, tt ^ 1)
                        gather(tt ^ 1, 0, buf ^ 1)
                else:
                    gather(tt, u + 1, buf ^ 1)

                gather_wait(tt, u, buf)
                scat_start(tt, u, buf)
        return 0

    lax.fori_loop(0, NSU // 2, body, 0)
    scat_wait(1, SUP - 1, 1)
    plsc.subcore_barrier()

    @pl.when(c == 0)
    def _():
        pltpu.sync_copy(acc_sh.at[pl.ds(s * RPT, RPT)],
                        p0_hbm.at[pl.ds(s * RPT, RPT)])

    @pl.when(c == 1)
    def _():
        pltpu.sync_copy(acc_sh.at[pl.ds(s * RPT, RPT)],
                        p1_hbm.at[pl.ds(s * RPT, RPT)])


_scat_call = functools.partial(
    pl.kernel,
    out_type=(
        jax.ShapeDtypeStruct((NP, D), jnp.float32),
        jax.ShapeDtypeStruct((NP, D), jnp.float32),
    ),
    mesh=_MESH,
    scratch_types=[
        pltpu.VMEM((2, SUP, CH), jnp.int32),
        pltpu.VMEM((2, SUP, CH), jnp.int32),
        pltpu.VMEM((2, CH, D), jnp.float32),
        pltpu.VMEM_SHARED((NP, D), jnp.float32),
        pltpu.SemaphoreType.DMA((2,)),
        pltpu.SemaphoreType.DMA((2,)),
        pltpu.SemaphoreType.DMA((2,)),
    ],
)(_scat_body)


def _comb_body(p0_ref, p1_ref, h2_ref, dinv_ref, b_ref, out_ref):
    out_ref[...] = (
        dinv_ref[...] * (p0_ref[...] + p1_ref[...] + h2_ref[...]) + b_ref[...]
    )


def _comb_call(p0, p1, h2, dinv, b2):
    grid = N // _MMR
    return pl.pallas_call(
        _comb_body,
        grid=(grid,),
        in_specs=[
            pl.BlockSpec((_MMR, D), lambda i: (i, 0)),
            pl.BlockSpec((_MMR, D), lambda i: (i, 0)),
            pl.BlockSpec((_MMR, D), lambda i: (i, 0)),
            pl.BlockSpec((_MMR, 1), lambda i: (i, 0)),
            pl.BlockSpec((1, D), lambda i: (0, 0)),
        ],
        out_specs=pl.BlockSpec((_MMR, D), lambda i: (i, 0)),
        out_shape=jax.ShapeDtypeStruct((N, D), jnp.float32),
    )(p0, p1, h2, dinv, b2)


def kernel(x, edge_index, W, b):
    row = edge_index[0]
    col = edge_index[1]
    row_p = jnp.concatenate([row, _ROW_PAD])
    col_p = jnp.concatenate([col, _COL_PAD])
    row3 = row_p.reshape(NW, CPT, CH)
    col3 = col_p.reshape(NW, CPT, CH)

    deg0, deg1 = _deg_call(col3)
    d0 = deg0.reshape(NP, 1)
    d1 = deg1.reshape(NP, 1)
    h2, dinv = _mm_call(x, W, d0, d1)
    p0, p1 = _scat_call(row3, col3, h2)
    b2 = b[None, :]
    return _comb_call(p0, p1, h2, dinv, b2)


# speedup vs baseline: 3.8912x; 1.0834x over previous
import functools

import jax
import jax.numpy as jnp
from jax import lax
from jax.experimental import pallas as pl
from jax.experimental.pallas import tpu as pltpu
from jax.experimental.pallas import tpu_sc as plsc

N = 10000
E = 320000
D = 128

NC, NS = 2, 16
NW = NC * NS
CH = 128
NCHUNK = E // CH
CPT = NCHUNK // NW
XTRA = NCHUNK - CPT * NW
NP = 10240
RPT = NP // NS
SUP = 3
NSU = CPT // SUP

_MESH = plsc.VectorSubcoreMesh(
    core_axis_name="c", subcore_axis_name="s", num_cores=NC, num_subcores=NS
)


def _worker():
    return lax.axis_index("s") * NC + lax.axis_index("c")


def _deg_body(edge_hbm, deg0_hbm, deg1_hbm, ones_v, cidx_v, zrow_v, deg_sh,
              semI, semS):
    c = lax.axis_index("c")
    s = lax.axis_index("s")
    w = _worker()

    def fill(i, _):
        ones_v[pl.ds(i * 16, 16)] = jnp.ones((16,), jnp.float32)
        zrow_v[pl.ds(i * 16, 16)] = jnp.zeros((16,), jnp.float32)
        return 0

    lax.fori_loop(0, CH // 16, fill, 0)

    def col_src(ch):
        return edge_hbm.at[1, pl.ds(pl.multiple_of(ch * CH, CH), CH)]

    def iload(j, _):
        pltpu.async_copy(col_src(w * CPT + j), cidx_v.at[j], semI)
        return 0

    lax.fori_loop(0, CPT, iload, 0)

    @pl.when(w < XTRA)
    def _():
        pltpu.async_copy(col_src(NW * CPT + w), cidx_v.at[CPT], semI)

    def zloop(i, _):
        pltpu.sync_copy(zrow_v, deg_sh.at[pl.ds(s * RPT + i * CH, CH)])
        return 0

    lax.fori_loop(0, RPT // CH, zloop, 0)

    def idrain(j, _):
        pltpu.make_async_copy(col_src(w * CPT + j), cidx_v.at[j], semI).wait()
        return 0

    lax.fori_loop(0, CPT, idrain, 0)

    @pl.when(w < XTRA)
    def _():
        pltpu.make_async_copy(col_src(NW * CPT + w), cidx_v.at[CPT],
                              semI).wait()

    plsc.subcore_barrier()

    def body(j, _):
        pltpu.async_copy(ones_v, deg_sh.at[cidx_v.at[j]], semS, add=True)
        return 0

    lax.fori_loop(0, CPT, body, 0)

    @pl.when(w < XTRA)
    def _():
        pltpu.async_copy(ones_v, deg_sh.at[cidx_v.at[CPT]], semS, add=True)

    def drain(j, _):
        pltpu.make_async_copy(ones_v, deg_sh.at[cidx_v.at[j]], semS).wait()
        return 0

    lax.fori_loop(0, CPT, drain, 0)

    @pl.when(w < XTRA)
    def _():
        pltpu.make_async_copy(ones_v, deg_sh.at[cidx_v.at[CPT]], semS).wait()

    plsc.subcore_barrier()

    @pl.when(c == 0)
    def _():
        pltpu.sync_copy(deg_sh.at[pl.ds(s * RPT, RPT)],
                        deg0_hbm.at[pl.ds(s * RPT, RPT)])

    @pl.when(c == 1)
    def _():
        pltpu.sync_copy(deg_sh.at[pl.ds(s * RPT, RPT)],
                        deg1_hbm.at[pl.ds(s * RPT, RPT)])


_deg_call = functools.partial(
    pl.kernel,
    out_type=(
        jax.ShapeDtypeStruct((NP,), jnp.float32),
        jax.ShapeDtypeStruct((NP,), jnp.float32),
    ),
    mesh=_MESH,
    scratch_types=[
        pltpu.VMEM((CH,), jnp.float32),
        pltpu.VMEM((CPT + 1, CH), jnp.int32),
        pltpu.VMEM((CH,), jnp.float32),
        pltpu.VMEM_SHARED((NP,), jnp.float32),
        pltpu.SemaphoreType.DMA,
        pltpu.SemaphoreType.DMA,
    ],
)(_deg_body)


def _mm_body(x_ref, w_ref, d0_ref, d1_ref, h2_ref, dinv_ref):
    deg = d0_ref[...] + d1_ref[...] + 1.0
    dinv = lax.rsqrt(deg)
    h = jnp.dot(x_ref[...], w_ref[...], preferred_element_type=jnp.float32)
    h2_ref[...] = h * dinv
    dinv_ref[...] = dinv


_MMR = 2000


def _mm_call(x, W, d0, d1):
    grid = N // _MMR
    return pl.pallas_call(
        _mm_body,
        grid=(grid,),
        in_specs=[
            pl.BlockSpec((_MMR, D), lambda i: (i, 0)),
            pl.BlockSpec((D, D), lambda i: (0, 0)),
            pl.BlockSpec((_MMR, 1), lambda i: (i, 0)),
            pl.BlockSpec((_MMR, 1), lambda i: (i, 0)),
        ],
        out_specs=[
            pl.BlockSpec((_MMR, D), lambda i: (i, 0)),
            pl.BlockSpec((_MMR, 1), lambda i: (i, 0)),
        ],
        out_shape=[
            jax.ShapeDtypeStruct((N, D), jnp.float32),
            jax.ShapeDtypeStruct((N, 1), jnp.float32),
        ],
    )(x, W, d0, d1)


def _scat_body(edge_hbm, h2_hbm, p0_hbm, p1_hbm,
               ridx_v, cidx_v, rows_v, acc_sh, semI, semG, semS):
    c = lax.axis_index("c")
    s = lax.axis_index("s")
    w = _worker()

    def row_src(ch):
        return edge_hbm.at[0, pl.ds(pl.multiple_of(ch * CH, CH), CH)]

    def col_src(ch):
        return edge_hbm.at[1, pl.ds(pl.multiple_of(ch * CH, CH), CH)]

    def load_idx(t, q):
        for u in range(SUP):
            ch = w * CPT + t * SUP + u
            pltpu.async_copy(row_src(ch), ridx_v.at[q, u], semI.at[q])
            pltpu.async_copy(col_src(ch), cidx_v.at[q, u], semI.at[q])

    def idx_wait(t, q):
        for u in range(SUP):
            ch = w * CPT + t * SUP + u
            pltpu.make_async_copy(row_src(ch), ridx_v.at[q, u],
                                  semI.at[q]).wait()
            pltpu.make_async_copy(col_src(ch), cidx_v.at[q, u],
                                  semI.at[q]).wait()

    load_idx(0, 0)
    load_idx(1, 1)

    def zb(i, _):
        def zb2(j, _):
            rows_v[0, i, pl.ds(j * 16, 16)] = jnp.zeros((16,), jnp.float32)
            return 0
        lax.fori_loop(0, D // 16, zb2, 0)
        return 0

    lax.fori_loop(0, CH, zb, 0)

    for k in range(RPT // CH):
        pltpu.sync_copy(rows_v.at[0], acc_sh.at[pl.ds(s * RPT + k * CH, CH)])
    plsc.subcore_barrier()

    def gather(q, u, buf):
        pltpu.async_copy(h2_hbm.at[ridx_v.at[q, u]], rows_v.at[buf],
                         semG.at[buf])

    def gather_wait(q, u, buf):
        pltpu.make_async_copy(h2_hbm.at[ridx_v.at[q, u]], rows_v.at[buf],
                              semG.at[buf]).wait()

    def scat_start(q, u, buf):
        pltpu.async_copy(rows_v.at[buf], acc_sh.at[cidx_v.at[q, u]],
                         semS.at[buf], add=True)

    def scat_wait(q, u, buf):
        pltpu.make_async_copy(rows_v.at[buf], acc_sh.at[cidx_v.at[q, u]],
                              semS.at[buf]).wait()

    idx_wait(0, 0)
    gather(0, 0, 0)

    def body(tp, _):
        for tt in range(2):
            t = tp * 2 + tt
            for u in range(SUP):
                j = t * SUP + u
                buf = (tt + u) % 2
                qm = (tt ^ 1, SUP - 1) if u == 0 else (tt, u - 1)

                @pl.when(j > 0)
                def _():
                    scat_wait(qm[0], qm[1], buf ^ 1)

                if u == 0:
                    @pl.when(jnp.logical_and(t >= 1, t + 1 < NSU))
                    def _():
                        load_idx(t + 1, tt ^ 1)

                if u == SUP - 1:
                    @pl.when(t + 1 < NSU)
                    def _():
                        idx_wait(t + 1, tt ^ 1)
                        gather(tt ^ 1, 0, buf ^ 1)
                else:
                    gather(tt, u + 1, buf ^ 1)

                gather_wait(tt, u, buf)
                scat_start(tt, u, buf)
        return 0

    lax.fori_loop(0, NSU // 2, body, 0)
    scat_wait(1, SUP - 1, (1 + SUP - 1) % 2)

    @pl.when(w < XTRA)
    def _():
        ch = NW * CPT + w
        pltpu.async_copy(row_src(ch), ridx_v.at[0, 0], semI.at[0])
        pltpu.async_copy(col_src(ch), cidx_v.at[0, 0], semI.at[0])
        pltpu.make_async_copy(row_src(ch), ridx_v.at[0, 0], semI.at[0]).wait()
        pltpu.make_async_copy(col_src(ch), cidx_v.at[0, 0], semI.at[0]).wait()
        pltpu.async_copy(h2_hbm.at[ridx_v.at[0, 0]], rows_v.at[0], semG.at[0])
        pltpu.make_async_copy(h2_hbm.at[ridx_v.at[0, 0]], rows_v.at[0],
                              semG.at[0]).wait()
        pltpu.sync_copy(rows_v.at[0], acc_sh.at[cidx_v.at[0, 0]], add=True)

    plsc.subcore_barrier()

    @pl.when(c == 0)
    def _():
        pltpu.sync_copy(acc_sh.at[pl.ds(s * RPT, RPT)],
                        p0_hbm.at[pl.ds(s * RPT, RPT)])

    @pl.when(c == 1)
    def _():
        pltpu.sync_copy(acc_sh.at[pl.ds(s * RPT, RPT)],
                        p1_hbm.at[pl.ds(s * RPT, RPT)])


_scat_call = functools.partial(
    pl.kernel,
    out_type=(
        jax.ShapeDtypeStruct((NP, D), jnp.float32),
        jax.ShapeDtypeStruct((NP, D), jnp.float32),
    ),
    mesh=_MESH,
    scratch_types=[
        pltpu.VMEM((2, SUP, CH), jnp.int32),
        pltpu.VMEM((2, SUP, CH), jnp.int32),
        pltpu.VMEM((2, CH, D), jnp.float32),
        pltpu.VMEM_SHARED((NP, D), jnp.float32),
        pltpu.SemaphoreType.DMA((2,)),
        pltpu.SemaphoreType.DMA((2,)),
        pltpu.SemaphoreType.DMA((2,)),
    ],
)(_scat_body)


def _comb_body(p0_ref, p1_ref, h2_ref, dinv_ref, b_ref, out_ref):
    out_ref[...] = (
        dinv_ref[...] * (p0_ref[...] + p1_ref[...] + h2_ref[...]) + b_ref[...]
    )


def _comb_call(p0, p1, h2, dinv, b2):
    grid = N // _MMR
    return pl.pallas_call(
        _comb_body,
        grid=(grid,),
        in_specs=[
            pl.BlockSpec((_MMR, D), lambda i: (i, 0)),
            pl.BlockSpec((_MMR, D), lambda i: (i, 0)),
            pl.BlockSpec((_MMR, D), lambda i: (i, 0)),
            pl.BlockSpec((_MMR, 1), lambda i: (i, 0)),
            pl.BlockSpec((1, D), lambda i: (0, 0)),
        ],
        out_specs=pl.BlockSpec((_MMR, D), lambda i: (i, 0)),
        out_shape=jax.ShapeDtypeStruct((N, D), jnp.float32),
    )(p0, p1, h2, dinv, b2)


def kernel(x, edge_index, W, b):
    deg0, deg1 = _deg_call(edge_index)
    d0 = deg0.reshape(NP, 1)
    d1 = deg1.reshape(NP, 1)
    h2, dinv = _mm_call(x, W, d0, d1)
    p0, p1 = _scat_call(edge_index, h2)
    b2 = b.reshape(1, D)
    return _comb_call(p0, p1, h2, dinv, b2)
